# baseline (speedup 1.0000x reference)
import jax
import jax.numpy as jnp
from jax import lax
from jax.experimental import pallas as pl
from jax.experimental.pallas import tpu as pltpu
from jax.experimental.pallas import tpu_sc as plsc


def _feat_copy(x, gen, N, G, D):
    BR = 1000
    NBX = N // BR
    NB = (N + G) // BR

    def body(x_ref, g_ref, o_ref):
        i = pl.program_id(0)

        @pl.when(i < NBX)
        def _():
            o_ref[...] = x_ref[...]

        @pl.when(i >= NBX)
        def _():
            o_ref[...] = g_ref[...]

    return pl.pallas_call(
        body,
        grid=(NB,),
        in_specs=[
            pl.BlockSpec((BR, D), lambda i: (jnp.minimum(i, NBX - 1), 0)),
            pl.BlockSpec((BR, D), lambda i: (jnp.maximum(i - NBX, 0), 0)),
        ],
        out_specs=pl.BlockSpec((BR, D), lambda i: (i, 0)),
        out_shape=jax.ShapeDtypeStruct((N + G, D), jnp.float32),
    )(x, gen)


def kernel(x, edge_index, tails, gen_feat, num_pred):
    N, D = x.shape
    E = edge_index.shape[1]
    T = tails.shape[0]
    P = gen_feat.shape[0] // T
    G = T * P
    W = E + G

    info = plsc.get_sparse_core_info()
    NC, NS = info.num_cores, info.num_subcores
    NW = NC * NS

    EC = E // NW
    GC = max(16, G // NW)
    while G % GC or GC % 16:
        GC += 1
    NACT = G // GC

    mesh = plsc.VectorSubcoreMesh(core_axis_name="c", subcore_axis_name="s")

    def body(edge_h, tails_h, edge_o, eb0, eb1, tails_v, rep_v, ids_v, s2, s3):
        wid = lax.axis_index("s") * NC + lax.axis_index("c")

        d0 = pltpu.async_copy(edge_h.at[pl.ds(wid * EC, EC)], eb0, s2)
        d1 = pltpu.async_copy(edge_h.at[pl.ds(E + wid * EC, EC)], eb1, s3)

        @pl.when(wid < NACT)
        def _gen():
            pltpu.sync_copy(tails_h, tails_v)
            c0 = wid * GC
            iota = lax.iota(jnp.int32, 16)
            for j in range(GC // 16):
                k = iota + (c0 + j * 16)
                rep_v[pl.ds(j * 16, 16)] = plsc.load_gather(tails_v, [k // P])
                ids_v[pl.ds(j * 16, 16)] = k + N
            pltpu.sync_copy(rep_v, edge_o.at[pl.ds(E + c0, GC)])
            pltpu.sync_copy(ids_v, edge_o.at[pl.ds(W + E + c0, GC)])

        d0.wait()
        o0 = pltpu.async_copy(eb0, edge_o.at[pl.ds(wid * EC, EC)], s2)
        d1.wait()
        o1 = pltpu.async_copy(eb1, edge_o.at[pl.ds(W + wid * EC, EC)], s3)
        o0.wait()
        o1.wait()

    run = pl.kernel(
        body,
        out_type=[
            jax.ShapeDtypeStruct((2 * W,), jnp.int32),
        ],
        mesh=mesh,
        scratch_types=[
            pltpu.VMEM((EC,), jnp.int32),
            pltpu.VMEM((EC,), jnp.int32),
            pltpu.VMEM((T,), jnp.int32),
            pltpu.VMEM((GC,), jnp.int32),
            pltpu.VMEM((GC,), jnp.int32),
            pltpu.SemaphoreType.DMA,
            pltpu.SemaphoreType.DMA,
        ],
        compiler_params=pltpu.CompilerParams(needs_layout_passes=False),
    )

    (edge_flat,) = run(edge_index.reshape(-1), tails)
    new_feat = _feat_copy(x, gen_feat.astype(jnp.float32), N, G, D)
    return (new_feat, edge_flat.reshape(2, W))
